# Initial kernel scaffold; baseline (speedup 1.0000x reference)
import jax
import jax.numpy as jnp
from jax.experimental import pallas as pl


def kernel(x, edge_index, batch, W1, b1, W2, b2, W3, b3, W4, b4, Wg, bg, Wl, bl, Wp, bp):
    raise NotImplementedError("write your pallas kernel here")


# speedup vs baseline: 9.6316x; 9.6316x over previous
import functools

import jax
import jax.numpy as jnp
from jax import lax
from jax.experimental import pallas as pl
from jax.experimental.pallas import tpu as pltpu
from jax.experimental.pallas import tpu_sc as plsc

N = 10000
NP = 10240
E = 160000
D = 256
H = 512
G = 64

NC = 2
NS = 16
B = 100
ROWS_PER_TILE = NP // NS
WB = 64
NWB = ROWS_PER_TILE // WB

def _get_mesh():
    return plsc.VectorSubcoreMesh(core_axis_name="c", subcore_axis_name="s",
                                  num_cores=NC, num_subcores=NS)


def _sc_deg(dst2, zinit, ones16):
    nbat = dst2.shape[1]

    @functools.partial(
        pl.kernel,
        out_type=jax.ShapeDtypeStruct((NC, NP, 128), jnp.float32),
        mesh=_get_mesh(),
        scratch_types=[
            pltpu.VMEM((nbat, B), jnp.int32),
            pltpu.VMEM((B, 128), jnp.float32),
            pltpu.VMEM((WB, 128), jnp.float32),
            pltpu.VMEM_SHARED((NP, 128), jnp.float32),
        ],
    )
    def k(dst_hbm, zinit_hbm, ones_hbm, out_hbm, idx_v, ones_v, stage_v, acc_sh):
        ci = lax.axis_index("c")
        si = lax.axis_index("s")
        w = ci * NS + si
        pltpu.sync_copy(dst_hbm.at[w], idx_v)
        pltpu.sync_copy(ones_hbm, ones_v)
        row0 = si * ROWS_PER_TILE
        for q in range(NWB):
            pltpu.sync_copy(zinit_hbm.at[pl.ds(row0 + q * WB, WB)], stage_v)
            pltpu.sync_copy(stage_v, acc_sh.at[pl.ds(row0 + q * WB, WB)])
        plsc.subcore_barrier()

        @pl.loop(0, nbat)
        def body(j):
            pltpu.sync_copy(ones_v, acc_sh.at[idx_v.at[j]], add=True)

        plsc.subcore_barrier()
        for q in range(NWB):
            pltpu.sync_copy(acc_sh.at[pl.ds(row0 + q * WB, WB)], stage_v)
            pltpu.sync_copy(stage_v, out_hbm.at[ci, pl.ds(row0 + q * WB, WB)])

    return k(dst2, zinit, ones16)


def _sc_spmm(y, src_r, dst_r):
    C = y.shape[0]
    F = y.shape[2]
    eb = src_r.shape[1]
    cpc = C // NC

    @functools.partial(
        pl.kernel,
        out_type=jax.ShapeDtypeStruct((C, NP, F), jnp.float32),
        mesh=_get_mesh(),
        scratch_types=[
            pltpu.VMEM((eb, B), jnp.int32),
            pltpu.VMEM((eb, B), jnp.int32),
            pltpu.VMEM((B, F), jnp.float32),
            pltpu.VMEM((WB, F), jnp.float32),
            pltpu.VMEM_SHARED((NP, F), jnp.float32),
        ],
    )
    def k(y_hbm, src_hbm, dst_hbm, z_hbm, src_v, dst_v, gbuf, stage_v, acc_sh):
        ci = lax.axis_index("c")
        si = lax.axis_index("s")
        pltpu.sync_copy(src_hbm.at[si], src_v)
        pltpu.sync_copy(dst_hbm.at[si], dst_v)
        row0 = si * ROWS_PER_TILE
        for cc in range(cpc):
            c = ci * cpc + cc
            for q in range(NWB):
                pltpu.sync_copy(y_hbm.at[c, pl.ds(row0 + q * WB, WB)], stage_v)
                pltpu.sync_copy(stage_v, acc_sh.at[pl.ds(row0 + q * WB, WB)])
            plsc.subcore_barrier()

            @pl.loop(0, eb)
            def body(j):
                pltpu.sync_copy(y_hbm.at[c].at[src_v.at[j]], gbuf)
                pltpu.sync_copy(gbuf, acc_sh.at[dst_v.at[j]], add=True)

            plsc.subcore_barrier()
            for q in range(NWB):
                pltpu.sync_copy(acc_sh.at[pl.ds(row0 + q * WB, WB)], stage_v)
                pltpu.sync_copy(stage_v, z_hbm.at[c, pl.ds(row0 + q * WB, WB)])
            plsc.subcore_barrier()

    return k(y, src_r, dst_r)


RB = 1024
NRB = NP // RB


def _tc_prep(deg2, x):
    C0 = D // 128

    def body(deg_ref, x_ref, dinv_ref, y_ref):
        deg = deg_ref[0, :, 0:1] + deg_ref[1, :, 0:1] + 1.0
        dv = 1.0 / jnp.sqrt(deg)
        dinv_ref[...] = dv
        xv = x_ref[...] * dv
        for c in range(C0):
            y_ref[c] = xv[:, c * 128:(c + 1) * 128]

    return pl.pallas_call(
        body,
        grid=(NRB,),
        in_specs=[
            pl.BlockSpec((2, RB, 128), lambda i: (0, i, 0)),
            pl.BlockSpec((RB, D), lambda i: (i, 0)),
        ],
        out_specs=[
            pl.BlockSpec((RB, 1), lambda i: (i, 0)),
            pl.BlockSpec((C0, RB, 128), lambda i: (0, i, 0)),
        ],
        out_shape=[
            jax.ShapeDtypeStruct((NP, 1), jnp.float32),
            jax.ShapeDtypeStruct((C0, NP, 128), jnp.float32),
        ],
    )(deg2, x)


def _tc_layer(z, dinv, Wr, b2, Wg_l):
    Cin = z.shape[0]
    Cout = H // 128

    def body(z_ref, dinv_ref, w_ref, b_ref, wg_ref, y_ref, g_ref):
        dv = dinv_ref[...]
        acc = jnp.zeros((RB, H), jnp.float32)
        for c in range(Cin):
            acc = acc + jnp.dot(z_ref[c] * dv, w_ref[c],
                                preferred_element_type=jnp.float32)
        x = jnp.maximum(acc + b_ref[...], 0.0)
        g_ref[...] = jnp.dot(x, wg_ref[...], preferred_element_type=jnp.float32)
        yv = x * dv
        for c in range(Cout):
            y_ref[c] = yv[:, c * 128:(c + 1) * 128]

    return pl.pallas_call(
        body,
        grid=(NRB,),
        in_specs=[
            pl.BlockSpec((Cin, RB, 128), lambda i: (0, i, 0)),
            pl.BlockSpec((RB, 1), lambda i: (i, 0)),
            pl.BlockSpec((Cin, 128, H), lambda i: (0, 0, 0)),
            pl.BlockSpec((1, H), lambda i: (0, 0)),
            pl.BlockSpec((H, 1), lambda i: (0, 0)),
        ],
        out_specs=[
            pl.BlockSpec((Cout, RB, 128), lambda i: (0, i, 0)),
            pl.BlockSpec((RB, 1), lambda i: (i, 0)),
        ],
        out_shape=[
            jax.ShapeDtypeStruct((Cout, NP, 128), jnp.float32),
            jax.ShapeDtypeStruct((NP, 1), jnp.float32),
        ],
    )(z, dinv, Wr, b2, Wg_l)


def _tc_pool(ys, gs, dinv, batch2, bg2, Wl, bl2, Wp, bp2):
    def body(y1_ref, y2_ref, y3_ref, y4_ref, g1_ref, g2_ref, g3_ref, g4_ref,
             dinv_ref, batch_ref, bg_ref, wl_ref, bl_ref, wp_ref, bp_ref,
             out_ref, m_s, den_s, pooled_s):
        j = pl.program_id(0)
        iota_g = lax.broadcasted_iota(jnp.int32, (1, G), 1)
        gblk = (g1_ref[...] + g2_ref[...] + g3_ref[...] + g4_ref[...]
                + bg_ref[...])
        oh = (batch_ref[...] == iota_g)

        @pl.when(j == 0)
        def _init():
            m_s[...] = jnp.full_like(m_s, -1e30)
            den_s[...] = jnp.zeros_like(den_s)
            pooled_s[...] = jnp.zeros_like(pooled_s)

        @pl.when(j < NRB)
        def _max_pass():
            bm = jnp.max(jnp.where(oh, gblk, -1e30), axis=0, keepdims=True)
            m_s[...] = jnp.maximum(m_s[...], bm)

        @pl.when(j >= NRB)
        def _acc_pass():
            mb = jnp.sum(jnp.where(oh, m_s[...], 0.0), axis=1, keepdims=True)
            ex = jnp.exp(gblk - mb)
            den_s[...] += jnp.sum(jnp.where(oh, ex, 0.0), axis=0, keepdims=True)
            ohbf = jnp.where(oh, 1.0, 0.0)
            w = ex / dinv_ref[...]
            contrib = []
            for y_ref in (y1_ref, y2_ref, y3_ref, y4_ref):
                for c in range(4):
                    contrib.append(lax.dot_general(
                        ohbf, y_ref[c] * w, (((0,), (0,)), ((), ())),
                        preferred_element_type=jnp.float32))
            pooled_s[...] += jnp.concatenate(contrib, axis=1)

        @pl.when(j == 2 * NRB - 1)
        def _head():
            den_c = jnp.maximum(den_s[...], 1e-16).reshape(G, 1)
            pooled = pooled_s[...] / den_c
            hg = jnp.maximum(
                jnp.dot(pooled, wl_ref[...],
                        preferred_element_type=jnp.float32) + bl_ref[...], 0.0)
            out_ref[...] = jnp.dot(hg, wp_ref[...],
                                   preferred_element_type=jnp.float32) + bp_ref[...]

    full = lambda shape: pl.BlockSpec(shape, lambda j: tuple(0 for _ in shape))
    blk = lambda j: (j % NRB, 0)
    y_spec = pl.BlockSpec((4, RB, 128),
                          lambda j: (0, jnp.maximum(j - NRB, 0), 0))
    col = pl.BlockSpec((RB, 1), blk)
    return pl.pallas_call(
        body,
        grid=(2 * NRB,),
        in_specs=[y_spec, y_spec, y_spec, y_spec,
                  col, col, col, col,
                  col, pl.BlockSpec((RB, 1), blk),
                  full((1, 1)), full((4 * H, H)), full((1, H)),
                  full((H, 2)), full((1, 2))],
        out_specs=pl.BlockSpec((G, 2), lambda j: (0, 0)),
        out_shape=jax.ShapeDtypeStruct((G, 2), jnp.float32),
        scratch_shapes=[
            pltpu.VMEM((1, G), jnp.float32),
            pltpu.VMEM((1, G), jnp.float32),
            pltpu.VMEM((G, 4 * H), jnp.float32),
        ],
    )(*ys, *gs, dinv, batch2, bg2, Wl, bl2, Wp, bp2)


def kernel(x, edge_index, batch, W1, b1, W2, b2, W3, b3, W4, b4,
           Wg, bg, Wl, bl, Wp, bp):
    src = edge_index[0].astype(jnp.int32)
    dst = edge_index[1].astype(jnp.int32)
    src_r = src.reshape(NS, E // (NS * B), B)
    dst_r = dst.reshape(NS, E // (NS * B), B)
    dst2 = dst.reshape(NC * NS, E // (NC * NS * B), B)
    zinit = jnp.zeros((NP, 128), jnp.float32)
    ones16 = jnp.zeros((B, 128), jnp.float32).at[:, 0].set(1.0)

    deg2 = _sc_deg(dst2, zinit, ones16)
    xp = jnp.zeros((NP, D), jnp.float32).at[:N].set(x)
    dinv, y0 = _tc_prep(deg2, xp)

    ys, gs = [], []
    y = y0
    params = [(W1.reshape(D // 128, 128, H), b1, Wg[0:H]),
              (W2.reshape(4, 128, H), b2, Wg[H:2 * H]),
              (W3.reshape(4, 128, H), b3, Wg[2 * H:3 * H]),
              (W4.reshape(4, 128, H), b4, Wg[3 * H:4 * H])]
    for Wr_l, b_l, Wg_l in params:
        z = _sc_spmm(y, src_r, dst_r)
        y, g_l = _tc_layer(z, dinv, Wr_l, b_l.reshape(1, H), Wg_l)
        ys.append(y)
        gs.append(g_l)

    batch2 = jnp.full((NP, 1), G, jnp.int32).at[:N, 0].set(batch.astype(jnp.int32))
    out = _tc_pool(ys, gs, dinv, batch2, bg.reshape(1, 1),
                   Wl, bl.reshape(1, H), Wp, bp.reshape(1, 2))
    return out
